# baseline (speedup 1.0000x reference)
from functools import partial

import jax
import jax.numpy as jnp
from jax.experimental import pallas as pl
from jax.experimental.pallas import tpu as pltpu


def _edge_kernel(e_ref, dt_ref, len_ref,
                 wd_ref, bd_ref, whxe_ref, whxa_ref, bge_ref, bga_ref,
                 attnw_ref, eoe_ref,
                 m_ref, a_ref, *, hidden, t_steps, din_e):
    H = hidden
    T = t_steps
    D = din_e
    TE = dt_ref.shape[0]
    f32 = jnp.float32
    bf16 = jnp.bfloat16

    dtm = dt_ref[...] - 1.0
    lens = len_ref[...]
    wd = wd_ref[...]
    whx_e = whxe_ref[...]
    whx_a = whxa_ref[...]
    bd = bd_ref[...]
    bg_e = bge_ref[...]
    bg_a = bga_ref[...]

    h = jnp.zeros((TE, 2 * H), f32)
    c = jnp.zeros((TE, 2 * H), f32)
    h_last = jnp.zeros((TE, 2 * H), f32)

    for s in range(T):
        x_s = e_ref[:, s, :].astype(bf16)
        c_s = jnp.tanh(
            jnp.dot(c.astype(bf16), wd, preferred_element_type=f32) + bd)
        c_adj = c + c_s * dtm[:, s:s + 1]
        h_bf = h.astype(bf16)
        lhs_e = jnp.concatenate([h_bf[:, :H], x_s], axis=1)
        lhs_a = jnp.concatenate([h_bf[:, H:], x_s], axis=1)
        ge = 0.5 * jnp.tanh(
            jnp.dot(lhs_e, whx_e, preferred_element_type=f32) + bg_e) + 0.5
        ga = 0.5 * jnp.tanh(
            jnp.dot(lhs_a, whx_a, preferred_element_type=f32) + bg_a) + 0.5
        f = jnp.concatenate([ge[:, 0:H], ga[:, 0:H]], axis=1)
        i = jnp.concatenate([ge[:, H:2 * H], ga[:, H:2 * H]], axis=1)
        o = jnp.concatenate([ge[:, 2 * H:3 * H], ga[:, 2 * H:3 * H]], axis=1)
        ct = jnp.concatenate([ge[:, 3 * H:4 * H], ga[:, 3 * H:4 * H]], axis=1)
        c = f * c_adj + i * ct
        h = o * jnp.tanh(c)
        h_last = jnp.where(lens > s, h, h_last)

    e_out = h_last[:, :H]
    a_hid = h_last[:, H:2 * H]

    a = jnp.dot(a_hid.astype(bf16), attnw_ref[...], preferred_element_type=f32)
    a = jnp.where(a > 0.0, a, 0.01 * a)

    m = jnp.dot(e_out.astype(bf16), eoe_ref[...], preferred_element_type=f32)

    m_ref[...] = m.astype(m_ref.dtype)
    a_ref[...] = a


def _pad_axis(x, size, axis):
    pad = size - x.shape[axis]
    if pad == 0:
        return x
    widths = [(0, 0)] * x.ndim
    widths[axis] = (0, pad)
    return jnp.pad(x, widths)


def _edge_messages(e3d, dt2, len2, fp, *, hidden, t_steps, din_e, tile=1000):
    E = e3d.shape[0]
    H = hidden
    n_blocks = int(pl.cdiv(E, tile))
    Ep = n_blocks * tile
    e3d = _pad_axis(e3d, Ep, 0)
    dt2 = _pad_axis(dt2, Ep, 0)
    len2 = _pad_axis(len2, Ep, 0)

    body = partial(_edge_kernel, hidden=H, t_steps=t_steps, din_e=din_e)
    m, a = pl.pallas_call(
        body,
        out_shape=[jax.ShapeDtypeStruct((Ep, H), jnp.bfloat16),
                   jax.ShapeDtypeStruct((Ep, 1), jnp.float32)],
        grid=(n_blocks,),
        in_specs=[
            pl.BlockSpec((tile, t_steps, din_e), lambda i: (i, 0, 0)),
            pl.BlockSpec((tile, t_steps), lambda i: (i, 0)),
            pl.BlockSpec((tile, 1), lambda i: (i, 0)),
            pl.BlockSpec((2 * H, 2 * H), lambda i: (0, 0)),
            pl.BlockSpec((1, 2 * H), lambda i: (0, 0)),
            pl.BlockSpec((H + din_e, 4 * H), lambda i: (0, 0)),
            pl.BlockSpec((H + din_e, 4 * H), lambda i: (0, 0)),
            pl.BlockSpec((1, 4 * H), lambda i: (0, 0)),
            pl.BlockSpec((1, 4 * H), lambda i: (0, 0)),
            pl.BlockSpec((H, 1), lambda i: (0, 0)),
            pl.BlockSpec((H, H), lambda i: (0, 0)),
        ],
        out_specs=[
            pl.BlockSpec((tile, H), lambda i: (i, 0)),
            pl.BlockSpec((tile, 1), lambda i: (i, 0)),
        ],
        compiler_params=pltpu.CompilerParams(dimension_semantics=("parallel",)),
    )(e3d, dt2, len2,
      fp["wd"], fp["bd"], fp["whx_e"], fp["whx_a"], fp["bg_e"], fp["bg_a"],
      fp["attn_w"], fp["eo_w_e"])
    return m[:E], a[:E]


def _reduce_kernel(a_ref, m_ref, src_ref, nid_ref, nf_ref,
                   eosrc_ref, eob_ref,
                   nusrc_ref, nuh_ref, nub_ref,
                   fcw_ref, fcb_ref, o_ref, *, k_deg, hidden):
    K = k_deg
    H = hidden
    f32 = jnp.float32
    bf16 = jnp.bfloat16
    NS = nf_ref.shape[0]
    z = a_ref[...]
    TD = z.shape[0]

    z = z - jnp.max(z, axis=1, keepdims=True)
    ksum = jnp.zeros((TD, K), f32)
    ssum = jnp.zeros((TD, K), f32)
    for r in range(K):
        zr = z if r == 0 else jnp.roll(z, r, axis=1)
        ge = (zr >= z).astype(f32)
        ksum = ksum + ge
        ssum = ssum + ge * zr
    in_sup = (1.0 + ksum * z > ssum).astype(f32)
    sk = jnp.sum(in_sup, axis=1, keepdims=True)
    sz = jnp.sum(in_sup * z, axis=1, keepdims=True)
    tau = (sz - 1.0) / sk
    alpha = jnp.maximum(z - tau, 0.0)

    nf_bf = nf_ref[...].astype(bf16)
    nfe = (jnp.dot(nf_bf, eosrc_ref[...], preferred_element_type=f32)
           + eob_ref[...]).astype(bf16)
    iota = jax.lax.broadcasted_iota(jnp.int32, (TD, NS), 1)
    src = src_ref[...]
    mp = m_ref[...]
    h_red = jnp.zeros((TD, H), f32)
    for j in range(K):
        oh_j = (iota == src[:, j:j + 1]).astype(bf16)
        hsm_j = jnp.dot(oh_j, nfe, preferred_element_type=f32)
        m_j = jnp.maximum(hsm_j + mp[:, j * H:(j + 1) * H].astype(f32), 0.0)
        h_red = h_red + alpha[:, j:j + 1] * m_j

    oh_n = (iota == nid_ref[...]).astype(bf16)
    self_h = jnp.dot(oh_n, nf_bf, preferred_element_type=f32)
    self_h_tmp = jnp.dot(oh_n, nfe, preferred_element_type=f32)
    hh = (h_red - self_h_tmp).astype(bf16)
    act = (jnp.dot(self_h, nusrc_ref[...], preferred_element_type=f32)
           + jnp.dot(hh, nuh_ref[...], preferred_element_type=f32)
           + nub_ref[...])
    act = jnp.maximum(act, 0.0).astype(bf16)
    out = jnp.dot(act, fcw_ref[...], preferred_element_type=f32) + fcb_ref[...]
    o_ref[...] = out


def _reduce_update(a_mb, m_mb, src_mb, nid2, nf, fp, *, hidden, k_deg,
                   tile=200):
    ND, K = a_mb.shape
    H = hidden
    ns, din_n = nf.shape
    C = fp["fc_w"].shape[1]
    n_blocks = int(pl.cdiv(ND, tile))
    NDp = n_blocks * tile
    a_mb = _pad_axis(a_mb, NDp, 0)
    m_mb = _pad_axis(m_mb, NDp, 0)
    src_mb = _pad_axis(src_mb, NDp, 0)
    nid2 = _pad_axis(nid2, NDp, 0)

    body = partial(_reduce_kernel, k_deg=K, hidden=H)
    out = pl.pallas_call(
        body,
        out_shape=jax.ShapeDtypeStruct((NDp, C), jnp.float32),
        grid=(n_blocks,),
        in_specs=[
            pl.BlockSpec((tile, K), lambda i: (i, 0)),
            pl.BlockSpec((tile, K * H), lambda i: (i, 0)),
            pl.BlockSpec((tile, K), lambda i: (i, 0)),
            pl.BlockSpec((tile, 1), lambda i: (i, 0)),
            pl.BlockSpec((ns, din_n), lambda i: (0, 0)),
            pl.BlockSpec((din_n, H), lambda i: (0, 0)),
            pl.BlockSpec((1, H), lambda i: (0, 0)),
            pl.BlockSpec((din_n, H), lambda i: (0, 0)),
            pl.BlockSpec((H, H), lambda i: (0, 0)),
            pl.BlockSpec((1, H), lambda i: (0, 0)),
            pl.BlockSpec((H, C), lambda i: (0, 0)),
            pl.BlockSpec((1, C), lambda i: (0, 0)),
        ],
        out_specs=pl.BlockSpec((tile, C), lambda i: (i, 0)),
        compiler_params=pltpu.CompilerParams(dimension_semantics=("parallel",)),
    )(a_mb, m_mb, src_mb, nid2, nf,
      fp["eo_w_src"], fp["eo_b"], fp["nu_w_self"], fp["nu_w_h"], fp["nu_b"],
      fp["fc_w"], fp["fc_b"])
    return out[:ND]


def kernel(wd, bd, wh, wx, bg, attn_w, eo_w_src, eo_w_e, eo_b,
           nu_w_self, nu_w_h, nu_b, fc_w, fc_b,
           node_features, edge_features, delta_t, edge_len, src_idx, layer_nid):
    H = 128
    bf16 = jnp.bfloat16
    n_dst, k = src_idx.shape
    n_edges, t_steps, din_e = edge_features.shape

    len2 = edge_len.reshape(n_edges, 1)
    nid2 = layer_nid.reshape(n_dst, 1)

    def _ecols(w):
        return jnp.concatenate(
            [w[:, 2 * g * H:(2 * g + 1) * H] for g in range(4)], axis=1)

    def _acols(w):
        return jnp.concatenate(
            [w[:, (2 * g + 1) * H:(2 * g + 2) * H] for g in range(4)], axis=1)

    whx_e = 0.5 * jnp.concatenate([_ecols(wh[:H]), _ecols(wx)], axis=0)
    whx_a = 0.5 * jnp.concatenate([_acols(wh[H:]), _acols(wx)], axis=0)
    fpA = {
        "wd": wd.astype(bf16), "bd": bd,
        "whx_e": whx_e.astype(bf16), "whx_a": whx_a.astype(bf16),
        "bg_e": 0.5 * _ecols(bg), "bg_a": 0.5 * _acols(bg),
        "attn_w": attn_w.astype(bf16),
        "eo_w_src": eo_w_src.astype(bf16), "eo_w_e": eo_w_e.astype(bf16),
        "eo_b": eo_b,
    }
    m_part, a = _edge_messages(edge_features, delta_t, len2, fpA,
                               hidden=H, t_steps=t_steps, din_e=din_e)

    a_mb = a.reshape(n_dst, k)
    m_mb = m_part.reshape(n_dst, k * H)

    fpB = {
        "eo_w_src": eo_w_src.astype(bf16), "eo_b": eo_b,
        "nu_w_self": nu_w_self, "nu_w_h": nu_w_h.astype(bf16), "nu_b": nu_b,
        "fc_w": fc_w.astype(bf16), "fc_b": fc_b,
    }
    return _reduce_update(a_mb, m_mb, src_idx, nid2, node_features, fpB,
                          hidden=H, k_deg=k)


# speedup vs baseline: 1.0104x; 1.0104x over previous
from functools import partial

import jax
import jax.numpy as jnp
from jax.experimental import pallas as pl
from jax.experimental.pallas import tpu as pltpu


def _edge_kernel(e_ref, dt_ref, len_ref,
                 wd_ref, bd_ref, whxe_ref, whxa_ref, bge_ref, bga_ref,
                 attnw_ref, eoe_ref,
                 m_ref, a_ref, *, hidden, t_steps, din_e):
    H = hidden
    T = t_steps
    D = din_e
    TE = dt_ref.shape[0]
    f32 = jnp.float32
    bf16 = jnp.bfloat16

    dtm = dt_ref[...] - 1.0
    lens = len_ref[...]
    wd = wd_ref[...]
    whx_e = whxe_ref[...]
    whx_a = whxa_ref[...]
    bd = bd_ref[...]
    bg_e = bge_ref[...]
    bg_a = bga_ref[...]

    h = jnp.zeros((TE, 2 * H), f32)
    c = jnp.zeros((TE, 2 * H), f32)
    h_last = jnp.zeros((TE, 2 * H), f32)

    for s in range(T):
        x_s = e_ref[:, s * D:(s + 1) * D].astype(bf16)
        c_s = jnp.tanh(
            jnp.dot(c.astype(bf16), wd, preferred_element_type=f32) + bd)
        c_adj = c + c_s * dtm[:, s:s + 1]
        h_bf = h.astype(bf16)
        lhs_e = jnp.concatenate([h_bf[:, :H], x_s], axis=1)
        lhs_a = jnp.concatenate([h_bf[:, H:], x_s], axis=1)
        ge = 0.5 * jnp.tanh(
            jnp.dot(lhs_e, whx_e, preferred_element_type=f32) + bg_e) + 0.5
        ga = 0.5 * jnp.tanh(
            jnp.dot(lhs_a, whx_a, preferred_element_type=f32) + bg_a) + 0.5
        f = jnp.concatenate([ge[:, 0:H], ga[:, 0:H]], axis=1)
        i = jnp.concatenate([ge[:, H:2 * H], ga[:, H:2 * H]], axis=1)
        o = jnp.concatenate([ge[:, 2 * H:3 * H], ga[:, 2 * H:3 * H]], axis=1)
        ct = jnp.concatenate([ge[:, 3 * H:4 * H], ga[:, 3 * H:4 * H]], axis=1)
        c = f * c_adj + i * ct
        h = o * jnp.tanh(c)
        h_last = jnp.where(lens > s, h, h_last)

    e_out = h_last[:, :H]
    a_hid = h_last[:, H:2 * H]

    a = jnp.dot(a_hid.astype(bf16), attnw_ref[...], preferred_element_type=f32)
    a = jnp.where(a > 0.0, a, 0.01 * a)

    m = jnp.dot(e_out.astype(bf16), eoe_ref[...], preferred_element_type=f32)

    m_ref[...] = m.astype(m_ref.dtype)
    a_ref[...] = a


def _pad_axis(x, size, axis):
    pad = size - x.shape[axis]
    if pad == 0:
        return x
    widths = [(0, 0)] * x.ndim
    widths[axis] = (0, pad)
    return jnp.pad(x, widths)


def _edge_messages(e2d, dt2, len2, fp, *, hidden, t_steps, din_e, tile=1000):
    E = e2d.shape[0]
    H = hidden
    n_blocks = int(pl.cdiv(E, tile))
    Ep = n_blocks * tile
    e2d = _pad_axis(e2d, Ep, 0)
    dt2 = _pad_axis(dt2, Ep, 0)
    len2 = _pad_axis(len2, Ep, 0)

    body = partial(_edge_kernel, hidden=H, t_steps=t_steps, din_e=din_e)
    m, a = pl.pallas_call(
        body,
        out_shape=[jax.ShapeDtypeStruct((Ep, H), jnp.bfloat16),
                   jax.ShapeDtypeStruct((Ep, 1), jnp.float32)],
        grid=(n_blocks,),
        in_specs=[
            pl.BlockSpec((tile, t_steps * din_e), lambda i: (i, 0)),
            pl.BlockSpec((tile, t_steps), lambda i: (i, 0)),
            pl.BlockSpec((tile, 1), lambda i: (i, 0)),
            pl.BlockSpec((2 * H, 2 * H), lambda i: (0, 0)),
            pl.BlockSpec((1, 2 * H), lambda i: (0, 0)),
            pl.BlockSpec((H + din_e, 4 * H), lambda i: (0, 0)),
            pl.BlockSpec((H + din_e, 4 * H), lambda i: (0, 0)),
            pl.BlockSpec((1, 4 * H), lambda i: (0, 0)),
            pl.BlockSpec((1, 4 * H), lambda i: (0, 0)),
            pl.BlockSpec((H, 1), lambda i: (0, 0)),
            pl.BlockSpec((H, H), lambda i: (0, 0)),
        ],
        out_specs=[
            pl.BlockSpec((tile, H), lambda i: (i, 0)),
            pl.BlockSpec((tile, 1), lambda i: (i, 0)),
        ],
        compiler_params=pltpu.CompilerParams(dimension_semantics=("parallel",)),
    )(e2d, dt2, len2,
      fp["wd"], fp["bd"], fp["whx_e"], fp["whx_a"], fp["bg_e"], fp["bg_a"],
      fp["attn_w"], fp["eo_w_e"])
    return m[:E], a[:E]


def _reduce_kernel(a_ref, m_ref, src_ref, nid_ref, nf_ref,
                   eosrc_ref, eob_ref,
                   nusrc_ref, nuh_ref, nub_ref,
                   fcw_ref, fcb_ref, o_ref, *, k_deg, hidden):
    K = k_deg
    H = hidden
    f32 = jnp.float32
    bf16 = jnp.bfloat16
    NS = nf_ref.shape[0]
    z = a_ref[...]
    TD = z.shape[0]

    z = z - jnp.max(z, axis=1, keepdims=True)
    ksum = jnp.zeros((TD, K), f32)
    ssum = jnp.zeros((TD, K), f32)
    for r in range(K):
        zr = z if r == 0 else jnp.roll(z, r, axis=1)
        ge = (zr >= z).astype(f32)
        ksum = ksum + ge
        ssum = ssum + ge * zr
    in_sup = (1.0 + ksum * z > ssum).astype(f32)
    sk = jnp.sum(in_sup, axis=1, keepdims=True)
    sz = jnp.sum(in_sup * z, axis=1, keepdims=True)
    tau = (sz - 1.0) / sk
    alpha = jnp.maximum(z - tau, 0.0)

    nf_bf = nf_ref[...].astype(bf16)
    nfe = (jnp.dot(nf_bf, eosrc_ref[...], preferred_element_type=f32)
           + eob_ref[...]).astype(bf16)
    iota = jax.lax.broadcasted_iota(jnp.int32, (TD, NS), 1)
    src = src_ref[...]
    mp = m_ref[...]
    h_red = jnp.zeros((TD, H), f32)
    for j in range(K):
        oh_j = (iota == src[:, j:j + 1]).astype(bf16)
        hsm_j = jnp.dot(oh_j, nfe, preferred_element_type=f32)
        m_j = jnp.maximum(hsm_j + mp[:, j * H:(j + 1) * H].astype(f32), 0.0)
        h_red = h_red + alpha[:, j:j + 1] * m_j

    oh_n = (iota == nid_ref[...]).astype(bf16)
    self_h = jnp.dot(oh_n, nf_bf, preferred_element_type=f32)
    self_h_tmp = jnp.dot(oh_n, nfe, preferred_element_type=f32)
    hh = (h_red - self_h_tmp).astype(bf16)
    act = (jnp.dot(self_h, nusrc_ref[...], preferred_element_type=f32)
           + jnp.dot(hh, nuh_ref[...], preferred_element_type=f32)
           + nub_ref[...])
    act = jnp.maximum(act, 0.0).astype(bf16)
    out = jnp.dot(act, fcw_ref[...], preferred_element_type=f32) + fcb_ref[...]
    o_ref[...] = out


def _reduce_update(a_mb, m_mb, src_mb, nid2, nf, fp, *, hidden, k_deg,
                   tile=200):
    ND, K = a_mb.shape
    H = hidden
    ns, din_n = nf.shape
    C = fp["fc_w"].shape[1]
    n_blocks = int(pl.cdiv(ND, tile))
    NDp = n_blocks * tile
    a_mb = _pad_axis(a_mb, NDp, 0)
    m_mb = _pad_axis(m_mb, NDp, 0)
    src_mb = _pad_axis(src_mb, NDp, 0)
    nid2 = _pad_axis(nid2, NDp, 0)

    body = partial(_reduce_kernel, k_deg=K, hidden=H)
    out = pl.pallas_call(
        body,
        out_shape=jax.ShapeDtypeStruct((NDp, C), jnp.float32),
        grid=(n_blocks,),
        in_specs=[
            pl.BlockSpec((tile, K), lambda i: (i, 0)),
            pl.BlockSpec((tile, K * H), lambda i: (i, 0)),
            pl.BlockSpec((tile, K), lambda i: (i, 0)),
            pl.BlockSpec((tile, 1), lambda i: (i, 0)),
            pl.BlockSpec((ns, din_n), lambda i: (0, 0)),
            pl.BlockSpec((din_n, H), lambda i: (0, 0)),
            pl.BlockSpec((1, H), lambda i: (0, 0)),
            pl.BlockSpec((din_n, H), lambda i: (0, 0)),
            pl.BlockSpec((H, H), lambda i: (0, 0)),
            pl.BlockSpec((1, H), lambda i: (0, 0)),
            pl.BlockSpec((H, C), lambda i: (0, 0)),
            pl.BlockSpec((1, C), lambda i: (0, 0)),
        ],
        out_specs=pl.BlockSpec((tile, C), lambda i: (i, 0)),
        compiler_params=pltpu.CompilerParams(dimension_semantics=("parallel",)),
    )(a_mb, m_mb, src_mb, nid2, nf,
      fp["eo_w_src"], fp["eo_b"], fp["nu_w_self"], fp["nu_w_h"], fp["nu_b"],
      fp["fc_w"], fp["fc_b"])
    return out[:ND]


def kernel(wd, bd, wh, wx, bg, attn_w, eo_w_src, eo_w_e, eo_b,
           nu_w_self, nu_w_h, nu_b, fc_w, fc_b,
           node_features, edge_features, delta_t, edge_len, src_idx, layer_nid):
    H = 128
    bf16 = jnp.bfloat16
    n_dst, k = src_idx.shape
    n_edges, t_steps, din_e = edge_features.shape

    e2d = edge_features.reshape(n_edges, t_steps * din_e)
    len2 = edge_len.reshape(n_edges, 1)
    nid2 = layer_nid.reshape(n_dst, 1)

    def _ecols(w):
        return jnp.concatenate(
            [w[:, 2 * g * H:(2 * g + 1) * H] for g in range(4)], axis=1)

    def _acols(w):
        return jnp.concatenate(
            [w[:, (2 * g + 1) * H:(2 * g + 2) * H] for g in range(4)], axis=1)

    whx_e = 0.5 * jnp.concatenate([_ecols(wh[:H]), _ecols(wx)], axis=0)
    whx_a = 0.5 * jnp.concatenate([_acols(wh[H:]), _acols(wx)], axis=0)
    fpA = {
        "wd": wd.astype(bf16), "bd": bd,
        "whx_e": whx_e.astype(bf16), "whx_a": whx_a.astype(bf16),
        "bg_e": 0.5 * _ecols(bg), "bg_a": 0.5 * _acols(bg),
        "attn_w": attn_w.astype(bf16),
        "eo_w_src": eo_w_src.astype(bf16), "eo_w_e": eo_w_e.astype(bf16),
        "eo_b": eo_b,
    }
    m_part, a = _edge_messages(e2d, delta_t, len2, fpA,
                               hidden=H, t_steps=t_steps, din_e=din_e)

    a_mb = a.reshape(n_dst, k)
    m_mb = m_part.reshape(n_dst, k * H)

    fpB = {
        "eo_w_src": eo_w_src.astype(bf16), "eo_b": eo_b,
        "nu_w_self": nu_w_self, "nu_w_h": nu_w_h.astype(bf16), "nu_b": nu_b,
        "fc_w": fc_w.astype(bf16), "fc_b": fc_b,
    }
    return _reduce_update(a_mb, m_mb, src_idx, nid2, node_features, fpB,
                          hidden=H, k_deg=k)
